# baseline (speedup 1.0000x reference)
import functools

import jax
import jax.numpy as jnp
from jax import lax
from jax.experimental import pallas as pl
from jax.experimental.pallas import tpu as pltpu
from jax.experimental.pallas import tpu_sc as plsc

N = 10000
E = 320000
F = 128
H = 64
G = 16
EPS = 1e-5

NC, NS = 2, 16
NW = NC * NS
NP = 10240
EP = 327680
BLK = 128
NBLK = EP // (NW * BLK)
DPT = E // NW
RB = 2048
NRB = NP // RB

@functools.cache
def _sc_params():
    import dataclasses
    cp = pltpu.CompilerParams()
    if "needs_layout_passes" in pltpu.CompilerParams.__dataclass_fields__:
        cp = dataclasses.replace(cp, needs_layout_passes=False)
    if "use_tc_tiling_on_sc" in pltpu.CompilerParams.__dataclass_fields__:
        cp = dataclasses.replace(cp, use_tc_tiling_on_sc=False)
    return cp


@functools.cache
def _mesh():
    return plsc.VectorSubcoreMesh(
        core_axis_name="c", subcore_axis_name="s",
        num_cores=NC, num_subcores=NS)


def _sc_deg_body(dst_hbm, idr_hbm, out_hbm, idx_v, hist_v, idr_v, zv, acc_sh,
                 sem):
    c = lax.axis_index("c")
    s = lax.axis_index("s")
    wid = s * NC + c
    zero16 = jnp.zeros((16,), jnp.float32)

    @pl.loop(0, 640)
    def _zero_hist(i):
        hist_v[i, :] = zero16

    @pl.loop(0, 40)
    def _zero_zv(i):
        zv[i, :] = zero16

    pltpu.sync_copy(zv, acc_sh.at[pl.ds(s * 40, 40)])
    pltpu.sync_copy(dst_hbm.at[pl.ds(wid * DPT, DPT)], idx_v)
    pltpu.sync_copy(idr_hbm, idr_v)
    plsc.subcore_barrier()

    ones16 = jnp.ones((16,), jnp.float32)

    @pl.loop(0, DPT // 16)
    def _hist(i):
        nid = idx_v[pl.ds(i * 16, 16)]
        row = lax.shift_right_logical(nid, 4)
        col = lax.bitwise_and(nid, 15)
        plsc.addupdate_scatter(hist_v, [row, col], ones16)

    @pl.loop(0, 5)
    def _merge(j):
        pltpu.sync_copy(hist_v.at[pl.ds(j * 128, 128)],
                        acc_sh.at[idr_v.at[j]], add=True)

    plsc.subcore_barrier()
    pltpu.sync_copy(acc_sh.at[pl.ds(s * 40, 40)],
                    out_hbm.at[pl.ds(c * 640 + s * 40, 40)])


@jax.jit
def _sc_deg(dst, idr):
    return pl.kernel(
        _sc_deg_body,
        out_type=jax.ShapeDtypeStruct((2 * 640, 16), jnp.float32),
        mesh=_mesh(),
        compiler_params=_sc_params(),
        scratch_types=[
            pltpu.VMEM((DPT,), jnp.int32),
            pltpu.VMEM((640, 16), jnp.float32),
            pltpu.VMEM((5, 128), jnp.int32),
            pltpu.VMEM((40, 16), jnp.float32),
            pltpu.VMEM_SHARED((640, 16), jnp.float32),
            pltpu.SemaphoreType.DMA,
        ],
    )(dst, idr)


def _sc_agg_body(tab_hbm, srcp_hbm, dstp_hbm, zer_hbm, out_hbm,
                 sidx_v, didx_v, rows, semg, sems, acc_sh):
    c = lax.axis_index("c")
    s = lax.axis_index("s")
    wid = s * NC + c
    rpt = NP // NS
    NB = 4

    pltpu.sync_copy(zer_hbm.at[pl.ds(s * rpt, rpt)],
                    acc_sh.at[pl.ds(s * rpt, rpt)])
    pltpu.sync_copy(srcp_hbm.at[wid], sidx_v)
    pltpu.sync_copy(dstp_hbm.at[wid], didx_v)
    plsc.subcore_barrier()

    def gather(k, b):
        pltpu.async_copy(tab_hbm.at[sidx_v.at[k]], rows.at[b], semg[b])

    def wait_gather(k, b):
        pltpu.make_async_copy(tab_hbm.at[sidx_v.at[k]], rows.at[b],
                              semg[b]).wait()

    def scat(k, b):
        pltpu.async_copy(rows.at[b], acc_sh.at[didx_v.at[k]], sems[b],
                         add=True)

    def wait_scat(k, b):
        pltpu.make_async_copy(rows.at[b], acc_sh.at[didx_v.at[k]],
                              sems[b]).wait()

    gather(0, 0)
    gather(1, 1)

    @pl.loop(0, NBLK, step=NB)
    def _edges(j):
        for b in range(NB):
            k = j + b
            gb = (b + 2) % NB

            @pl.when(k - 2 >= 0)
            def _():
                wait_scat(k - 2, gb)

            @pl.when(k + 2 < NBLK)
            def _():
                gather(k + 2, gb)

            wait_gather(k, b)
            scat(k, b)

    for k in (NBLK - 2, NBLK - 1):
        wait_scat(k, k % NB)

    plsc.subcore_barrier()
    pltpu.sync_copy(acc_sh.at[pl.ds(s * rpt, rpt)],
                    out_hbm.at[pl.ds(c * NP + s * rpt, rpt)])


@jax.jit
def _sc_agg(tab, srcp, dstp, zer):
    return pl.kernel(
        _sc_agg_body,
        out_type=jax.ShapeDtypeStruct((2 * NP, H), jnp.float32),
        mesh=_mesh(),
        compiler_params=_sc_params(),
        scratch_types=[
            pltpu.VMEM((NBLK, BLK), jnp.int32),
            pltpu.VMEM((NBLK, BLK), jnp.int32),
            pltpu.VMEM((4, BLK, H), jnp.float32),
            [pltpu.SemaphoreType.DMA] * 4,
            [pltpu.SemaphoreType.DMA] * 4,
            pltpu.VMEM_SHARED((NP, H), jnp.float32),
        ],
    )(tab, srcp, dstp, zer)


def _mm_body(x_ref, w_ref, o_ref):
    o_ref[...] = jnp.dot(x_ref[...], w_ref[...],
                         preferred_element_type=jnp.float32)


def _tc_mm(x, w):
    m, k = x.shape
    _, n = w.shape
    return pl.pallas_call(
        _mm_body,
        grid=(m // RB,),
        in_specs=[pl.BlockSpec((RB, k), lambda i: (i, 0)),
                  pl.BlockSpec((k, n), lambda i: (0, 0))],
        out_specs=pl.BlockSpec((RB, n), lambda i: (i, 0)),
        out_shape=jax.ShapeDtypeStruct((m, n), jnp.float32),
    )(x, w)


def _dinv_body(dp_ref, o_ref):
    deg = dp_ref[0] + dp_ref[1] + 1.0
    r = lax.broadcasted_iota(jnp.int32, (80, 128), 0)
    cidx = lax.broadcasted_iota(jnp.int32, (80, 128), 1)
    nid = r * 128 + cidx
    o_ref[...] = jnp.where(nid < N, lax.rsqrt(deg), 0.0)


def _tc_dinv(degp):
    return pl.pallas_call(
        _dinv_body,
        out_shape=jax.ShapeDtypeStruct((80, 128), jnp.float32),
    )(degp)


def _scale_body(m_ref, d_ref, o_ref):
    o_ref[...] = m_ref[...] * d_ref[...]


def _tc_scale(m, dcol):
    return pl.pallas_call(
        _scale_body,
        grid=(NRB,),
        in_specs=[pl.BlockSpec((RB, H), lambda i: (i, 0)),
                  pl.BlockSpec((RB, 1), lambda i: (i, 0))],
        out_specs=pl.BlockSpec((RB, H), lambda i: (i, 0)),
        out_shape=jax.ShapeDtypeStruct((NP, H), jnp.float32),
    )(m, dcol)


def _post_body(ap_ref, hp_ref, d_ref, prm_ref, o_ref, st_ref, *, brow):
    i = pl.program_id(0)
    b = prm_ref[brow:brow + 1, :H]
    v = (ap_ref[0] + ap_ref[1] + hp_ref[...]) * d_ref[...] + b
    rid = lax.broadcasted_iota(jnp.int32, (RB, 1), 0) + i * RB
    v = jnp.where(rid < N, v, 0.0)
    o_ref[...] = v
    srow = jnp.sum(v, axis=0, keepdims=True)
    qrow = jnp.sum(v * v, axis=0, keepdims=True)
    st = jnp.concatenate(
        [srow, qrow, jnp.zeros((6, H), jnp.float32)], axis=0)

    @pl.when(i == 0)
    def _():
        st_ref[...] = st

    @pl.when(i > 0)
    def _():
        st_ref[...] = st_ref[...] + st


def _tc_post(accp, hp, dcol, prm, brow):
    return pl.pallas_call(
        functools.partial(_post_body, brow=brow),
        grid=(NRB,),
        in_specs=[pl.BlockSpec((2, RB, H), lambda i: (0, i, 0)),
                  pl.BlockSpec((RB, H), lambda i: (i, 0)),
                  pl.BlockSpec((RB, 1), lambda i: (i, 0)),
                  pl.BlockSpec((8, 128), lambda i: (0, 0))],
        out_specs=[pl.BlockSpec((RB, H), lambda i: (i, 0)),
                   pl.BlockSpec((8, H), lambda i: (0, 0))],
        out_shape=[jax.ShapeDtypeStruct((NP, H), jnp.float32),
                   jax.ShapeDtypeStruct((8, H), jnp.float32)],
    )(accp, hp, dcol, prm)


def _bnmm_body(o1_ref, st_ref, prm_ref, w_ref, d_ref, o_ref, *, grow, berow):
    mu = st_ref[0:1, :] * (1.0 / N)
    var = st_ref[1:2, :] * (1.0 / N) - mu * mu
    istd = lax.rsqrt(var + EPS)
    g = prm_ref[grow:grow + 1, :H]
    be = prm_ref[berow:berow + 1, :H]
    h = jnp.maximum((o1_ref[...] - mu) * istd * g + be, 0.0)
    o_ref[...] = jnp.dot(h, w_ref[...],
                         preferred_element_type=jnp.float32) * d_ref[...]


def _tc_bnmm(o1, st, prm, w, dcol, grow, berow):
    return pl.pallas_call(
        functools.partial(_bnmm_body, grow=grow, berow=berow),
        grid=(NRB,),
        in_specs=[pl.BlockSpec((RB, H), lambda i: (i, 0)),
                  pl.BlockSpec((8, H), lambda i: (0, 0)),
                  pl.BlockSpec((8, 128), lambda i: (0, 0)),
                  pl.BlockSpec((H, H), lambda i: (0, 0)),
                  pl.BlockSpec((RB, 1), lambda i: (i, 0))],
        out_specs=pl.BlockSpec((RB, H), lambda i: (i, 0)),
        out_shape=jax.ShapeDtypeStruct((NP, H), jnp.float32),
    )(o1, st, prm, w, dcol)


def _final_body(o2_ref, st_ref, prm_ref, bt_ref, o_ref, cnt_ref):
    i = pl.program_id(0)
    mu = st_ref[0:1, :] * (1.0 / N)
    var = st_ref[1:2, :] * (1.0 / N) - mu * mu
    istd = lax.rsqrt(var + EPS)
    g = prm_ref[4:5, :H]
    be = prm_ref[5:6, :H]
    h = jnp.maximum((o2_ref[...] - mu) * istd * g + be, 0.0)
    b = bt_ref[0, 0, :]
    gid = lax.broadcasted_iota(jnp.int32, (G, RB), 0)
    oh = jnp.where(gid == b[None, :], 1.0, 0.0)
    ps = jnp.dot(oh, h, preferred_element_type=jnp.float32)
    cnt = jnp.broadcast_to(jnp.sum(oh, axis=1, keepdims=True), (G, H))

    @pl.when(i == 0)
    def _():
        o_ref[...] = ps
        cnt_ref[...] = cnt

    @pl.when(i > 0)
    def _():
        o_ref[...] = o_ref[...] + ps
        cnt_ref[...] = cnt_ref[...] + cnt

    @pl.when(i == NRB - 1)
    def _():
        o_ref[...] = o_ref[...] / jnp.maximum(cnt_ref[...], 1.0)


def _tc_final(o2, st, prm, bt):
    return pl.pallas_call(
        _final_body,
        grid=(NRB,),
        in_specs=[pl.BlockSpec((RB, H), lambda i: (i, 0)),
                  pl.BlockSpec((8, H), lambda i: (0, 0)),
                  pl.BlockSpec((8, 128), lambda i: (0, 0)),
                  pl.BlockSpec((1, 1, RB), lambda i: (i, 0, 0))],
        out_specs=pl.BlockSpec((G, H), lambda i: (0, 0)),
        out_shape=jax.ShapeDtypeStruct((G, H), jnp.float32),
        scratch_shapes=[pltpu.VMEM((G, H), jnp.float32)],
    )(o2, st, prm, bt)


def kernel(x, ei, batch, W1, b1, g1, be1, W2, b2, g2, be2):
    src = ei[0].astype(jnp.int32)
    dst = ei[1].astype(jnp.int32)
    srcp = jnp.concatenate(
        [src, jnp.zeros((EP - E,), jnp.int32)]).reshape(NW, NBLK, BLK)
    dstp = jnp.concatenate(
        [dst, jnp.full((EP - E,), N, jnp.int32)]).reshape(NW, NBLK, BLK)
    x_pad = jnp.pad(x, ((0, NP - N), (0, 0)))
    bt = jnp.concatenate(
        [batch.astype(jnp.int32),
         jnp.full((NP - N,), G, jnp.int32)]).reshape(NRB, 1, RB)
    idr = jnp.arange(640, dtype=jnp.int32).reshape(5, 128)
    zer = jnp.zeros((NP, H), jnp.float32)
    prm = jnp.pad(jnp.stack([b1, g1, be1, b2, g2, be2,
                             jnp.zeros_like(b1), jnp.zeros_like(b1)]),
                  ((0, 0), (0, 128 - H)))

    degp = _sc_deg(dst, idr)
    mm1 = _tc_mm(x_pad, W1)
    dinv = _tc_dinv(degp.reshape(2, 80, 128))
    dcol = dinv.reshape(NP)[:, None]

    h1p = _tc_scale(mm1, dcol)
    acc1 = _sc_agg(h1p, srcp, dstp, zer).reshape(2, NP, H)
    out1, st1 = _tc_post(acc1, h1p, dcol, prm, brow=0)
    h2p = _tc_bnmm(out1, st1, prm, W2, dcol, grow=1, berow=2)
    acc2 = _sc_agg(h2p, srcp, dstp, zer).reshape(2, NP, H)
    out2, st2 = _tc_post(acc2, h2p, dcol, prm, brow=3)
    return _tc_final(out2, st2, prm, bt)


# speedup vs baseline: 1.9265x; 1.9265x over previous
import functools

import jax
import jax.numpy as jnp
from jax import lax
from jax.experimental import pallas as pl
from jax.experimental.pallas import tpu as pltpu
from jax.experimental.pallas import tpu_sc as plsc

N = 10000
E = 320000
F = 128
H = 64
G = 16
EPS = 1e-5

NC, NS = 2, 16
NW = NC * NS
NP = 10240
EP = 327680
BLK = 128
NBLK = EP // (NW * BLK)
DPT = E // NW
RB = 2048
NRB = NP // RB

@functools.cache
def _sc_params():
    import dataclasses
    cp = pltpu.CompilerParams()
    if "needs_layout_passes" in pltpu.CompilerParams.__dataclass_fields__:
        cp = dataclasses.replace(cp, needs_layout_passes=False)
    if "use_tc_tiling_on_sc" in pltpu.CompilerParams.__dataclass_fields__:
        cp = dataclasses.replace(cp, use_tc_tiling_on_sc=False)
    return cp


@functools.cache
def _mesh():
    return plsc.VectorSubcoreMesh(
        core_axis_name="c", subcore_axis_name="s",
        num_cores=NC, num_subcores=NS)


def _sc_deg_body(dst_hbm, idr_hbm, out_hbm, idx_v, hist_v, idr_v, zv, acc_sh,
                 sem):
    c = lax.axis_index("c")
    s = lax.axis_index("s")
    wid = s * NC + c
    zero16 = jnp.zeros((16,), jnp.float32)

    @pl.loop(0, 640)
    def _zero_hist(i):
        hist_v[i, :] = zero16

    @pl.loop(0, 40)
    def _zero_zv(i):
        zv[i, :] = zero16

    pltpu.sync_copy(zv, acc_sh.at[pl.ds(s * 40, 40)])
    pltpu.sync_copy(dst_hbm.at[pl.ds(wid * DPT, DPT)], idx_v)
    pltpu.sync_copy(idr_hbm, idr_v)
    plsc.subcore_barrier()

    ones16 = jnp.ones((16,), jnp.float32)

    @pl.loop(0, DPT // 16)
    def _hist(i):
        nid = idx_v[pl.ds(i * 16, 16)]
        row = lax.shift_right_logical(nid, 4)
        col = lax.bitwise_and(nid, 15)
        plsc.addupdate_scatter(hist_v, [row, col], ones16)

    @pl.loop(0, 5)
    def _merge(j):
        pltpu.sync_copy(hist_v.at[pl.ds(j * 128, 128)],
                        acc_sh.at[idr_v.at[j]], add=True)

    plsc.subcore_barrier()
    pltpu.sync_copy(acc_sh.at[pl.ds(s * 40, 40)],
                    out_hbm.at[pl.ds(c * 640 + s * 40, 40)])


@jax.jit
def _sc_deg(dst, idr):
    return pl.kernel(
        _sc_deg_body,
        out_type=jax.ShapeDtypeStruct((2 * 640, 16), jnp.float32),
        mesh=_mesh(),
        compiler_params=_sc_params(),
        scratch_types=[
            pltpu.VMEM((DPT,), jnp.int32),
            pltpu.VMEM((640, 16), jnp.float32),
            pltpu.VMEM((5, 128), jnp.int32),
            pltpu.VMEM((40, 16), jnp.float32),
            pltpu.VMEM_SHARED((640, 16), jnp.float32),
            pltpu.SemaphoreType.DMA,
        ],
    )(dst, idr)


HH = H // 2


def _sc_agg_body(tabA, tabB, srcp_hbm, dstp_hbm, outA, outB,
                 sidx_v, didx_v, rows, semg, sems, acc_sh, tab_sh):
    c = lax.axis_index("c")
    s = lax.axis_index("s")
    wid = s * NC + c
    rpt = NP // NS
    NB = 4

    pltpu.sync_copy(srcp_hbm.at[wid], sidx_v)
    pltpu.sync_copy(dstp_hbm.at[wid], didx_v)

    for tab_hbm, out_hbm in ((tabA, outA), (tabB, outB)):
        pltpu.sync_copy(tab_hbm.at[pl.ds(s * rpt, rpt)],
                        tab_sh.at[pl.ds(s * rpt, rpt)])

        @pl.loop(0, BLK)
        def _zrow(i):
            @pl.loop(0, HH, step=16)
            def _zcol(k):
                rows[0, i, pl.ds(k, 16)] = jnp.zeros((16,), jnp.float32)

        @pl.loop(0, rpt, step=BLK)
        def _zacc(r):
            pltpu.sync_copy(rows.at[0], acc_sh.at[pl.ds(s * rpt + r, BLK)])

        plsc.subcore_barrier()

        def gather(k, b):
            pltpu.async_copy(tab_sh.at[sidx_v.at[k]], rows.at[b], semg[b])

        def wait_gather(k, b):
            pltpu.make_async_copy(tab_sh.at[sidx_v.at[k]], rows.at[b],
                                  semg[b]).wait()

        def scat(k, b):
            pltpu.async_copy(rows.at[b], acc_sh.at[didx_v.at[k]], sems[b],
                             add=True)

        def wait_scat(k, b):
            pltpu.make_async_copy(rows.at[b], acc_sh.at[didx_v.at[k]],
                                  sems[b]).wait()

        gather(0, 0)
        gather(1, 1)

        @pl.loop(0, NBLK, step=NB)
        def _edges(j):
            for b in range(NB):
                k = j + b
                gb = (b + 2) % NB

                @pl.when(k - 2 >= 0)
                def _():
                    wait_scat(k - 2, gb)

                @pl.when(k + 2 < NBLK)
                def _():
                    gather(k + 2, gb)

                wait_gather(k, b)
                scat(k, b)

        for k in (NBLK - 2, NBLK - 1):
            wait_scat(k, k % NB)

        plsc.subcore_barrier()
        pltpu.sync_copy(acc_sh.at[pl.ds(s * rpt, rpt)],
                        out_hbm.at[pl.ds(c * NP + s * rpt, rpt)])
        plsc.subcore_barrier()


@jax.jit
def _sc_agg(tabA, tabB, srcp, dstp):
    return pl.kernel(
        _sc_agg_body,
        out_type=(jax.ShapeDtypeStruct((2 * NP, HH), jnp.float32),
                  jax.ShapeDtypeStruct((2 * NP, HH), jnp.float32)),
        mesh=_mesh(),
        compiler_params=_sc_params(),
        scratch_types=[
            pltpu.VMEM((NBLK, BLK), jnp.int32),
            pltpu.VMEM((NBLK, BLK), jnp.int32),
            pltpu.VMEM((4, BLK, HH), jnp.float32),
            [pltpu.SemaphoreType.DMA] * 4,
            [pltpu.SemaphoreType.DMA] * 4,
            pltpu.VMEM_SHARED((NP, HH), jnp.float32),
            pltpu.VMEM_SHARED((NP, HH), jnp.float32),
        ],
    )(tabA, tabB, srcp, dstp)


def _mm_body(x_ref, w_ref, o_ref):
    o_ref[...] = jnp.dot(x_ref[...], w_ref[...],
                         preferred_element_type=jnp.float32)


def _tc_mm(x, w):
    m, k = x.shape
    _, n = w.shape
    return pl.pallas_call(
        _mm_body,
        grid=(m // RB,),
        in_specs=[pl.BlockSpec((RB, k), lambda i: (i, 0)),
                  pl.BlockSpec((k, n), lambda i: (0, 0))],
        out_specs=pl.BlockSpec((RB, n), lambda i: (i, 0)),
        out_shape=jax.ShapeDtypeStruct((m, n), jnp.float32),
    )(x, w)


def _dinv_body(dp_ref, o_ref):
    deg = dp_ref[0] + dp_ref[1] + 1.0
    r = lax.broadcasted_iota(jnp.int32, (80, 128), 0)
    cidx = lax.broadcasted_iota(jnp.int32, (80, 128), 1)
    nid = r * 128 + cidx
    o_ref[...] = jnp.where(nid < N, lax.rsqrt(deg), 0.0)


def _tc_dinv(degp):
    return pl.pallas_call(
        _dinv_body,
        out_shape=jax.ShapeDtypeStruct((80, 128), jnp.float32),
    )(degp)


def _scale_body(m_ref, d_ref, o_ref):
    o_ref[...] = m_ref[...] * d_ref[...]


def _tc_scale(m, dcol):
    return pl.pallas_call(
        _scale_body,
        grid=(NRB,),
        in_specs=[pl.BlockSpec((RB, H), lambda i: (i, 0)),
                  pl.BlockSpec((RB, 1), lambda i: (i, 0))],
        out_specs=pl.BlockSpec((RB, H), lambda i: (i, 0)),
        out_shape=jax.ShapeDtypeStruct((NP, H), jnp.float32),
    )(m, dcol)


def _post_body(apa_ref, apb_ref, hp_ref, d_ref, prm_ref, o_ref, st_ref, *,
               brow):
    i = pl.program_id(0)
    b = prm_ref[brow:brow + 1, :H]
    agg = jnp.concatenate(
        [apa_ref[0] + apa_ref[1], apb_ref[0] + apb_ref[1]], axis=1)
    v = (agg + hp_ref[...]) * d_ref[...] + b
    rid = lax.broadcasted_iota(jnp.int32, (RB, 1), 0) + i * RB
    v = jnp.where(rid < N, v, 0.0)
    o_ref[...] = v
    srow = jnp.sum(v, axis=0, keepdims=True)
    qrow = jnp.sum(v * v, axis=0, keepdims=True)
    st = jnp.concatenate(
        [srow, qrow, jnp.zeros((6, H), jnp.float32)], axis=0)

    @pl.when(i == 0)
    def _():
        st_ref[...] = st

    @pl.when(i > 0)
    def _():
        st_ref[...] = st_ref[...] + st


def _tc_post(accpa, accpb, hp, dcol, prm, brow):
    return pl.pallas_call(
        functools.partial(_post_body, brow=brow),
        grid=(NRB,),
        in_specs=[pl.BlockSpec((2, RB, HH), lambda i: (0, i, 0)),
                  pl.BlockSpec((2, RB, HH), lambda i: (0, i, 0)),
                  pl.BlockSpec((RB, H), lambda i: (i, 0)),
                  pl.BlockSpec((RB, 1), lambda i: (i, 0)),
                  pl.BlockSpec((8, 128), lambda i: (0, 0))],
        out_specs=[pl.BlockSpec((RB, H), lambda i: (i, 0)),
                   pl.BlockSpec((8, H), lambda i: (0, 0))],
        out_shape=[jax.ShapeDtypeStruct((NP, H), jnp.float32),
                   jax.ShapeDtypeStruct((8, H), jnp.float32)],
    )(accpa, accpb, hp, dcol, prm)


def _bnmm_body(o1_ref, st_ref, prm_ref, w_ref, d_ref, o_ref, *, grow, berow):
    mu = st_ref[0:1, :] * (1.0 / N)
    var = st_ref[1:2, :] * (1.0 / N) - mu * mu
    istd = lax.rsqrt(var + EPS)
    g = prm_ref[grow:grow + 1, :H]
    be = prm_ref[berow:berow + 1, :H]
    h = jnp.maximum((o1_ref[...] - mu) * istd * g + be, 0.0)
    o_ref[...] = jnp.dot(h, w_ref[...],
                         preferred_element_type=jnp.float32) * d_ref[...]


def _tc_bnmm(o1, st, prm, w, dcol, grow, berow):
    return pl.pallas_call(
        functools.partial(_bnmm_body, grow=grow, berow=berow),
        grid=(NRB,),
        in_specs=[pl.BlockSpec((RB, H), lambda i: (i, 0)),
                  pl.BlockSpec((8, H), lambda i: (0, 0)),
                  pl.BlockSpec((8, 128), lambda i: (0, 0)),
                  pl.BlockSpec((H, H), lambda i: (0, 0)),
                  pl.BlockSpec((RB, 1), lambda i: (i, 0))],
        out_specs=pl.BlockSpec((RB, H), lambda i: (i, 0)),
        out_shape=jax.ShapeDtypeStruct((NP, H), jnp.float32),
    )(o1, st, prm, w, dcol)


def _final_body(o2_ref, st_ref, prm_ref, bt_ref, o_ref, cnt_ref):
    i = pl.program_id(0)
    mu = st_ref[0:1, :] * (1.0 / N)
    var = st_ref[1:2, :] * (1.0 / N) - mu * mu
    istd = lax.rsqrt(var + EPS)
    g = prm_ref[4:5, :H]
    be = prm_ref[5:6, :H]
    h = jnp.maximum((o2_ref[...] - mu) * istd * g + be, 0.0)
    b = bt_ref[0, 0, :]
    gid = lax.broadcasted_iota(jnp.int32, (G, RB), 0)
    oh = jnp.where(gid == b[None, :], 1.0, 0.0)
    ps = jnp.dot(oh, h, preferred_element_type=jnp.float32)
    cnt = jnp.broadcast_to(jnp.sum(oh, axis=1, keepdims=True), (G, H))

    @pl.when(i == 0)
    def _():
        o_ref[...] = ps
        cnt_ref[...] = cnt

    @pl.when(i > 0)
    def _():
        o_ref[...] = o_ref[...] + ps
        cnt_ref[...] = cnt_ref[...] + cnt

    @pl.when(i == NRB - 1)
    def _():
        o_ref[...] = o_ref[...] / jnp.maximum(cnt_ref[...], 1.0)


def _tc_final(o2, st, prm, bt):
    return pl.pallas_call(
        _final_body,
        grid=(NRB,),
        in_specs=[pl.BlockSpec((RB, H), lambda i: (i, 0)),
                  pl.BlockSpec((8, H), lambda i: (0, 0)),
                  pl.BlockSpec((8, 128), lambda i: (0, 0)),
                  pl.BlockSpec((1, 1, RB), lambda i: (i, 0, 0))],
        out_specs=pl.BlockSpec((G, H), lambda i: (0, 0)),
        out_shape=jax.ShapeDtypeStruct((G, H), jnp.float32),
        scratch_shapes=[pltpu.VMEM((G, H), jnp.float32)],
    )(o2, st, prm, bt)


def kernel(x, ei, batch, W1, b1, g1, be1, W2, b2, g2, be2):
    src = ei[0].astype(jnp.int32)
    dst = ei[1].astype(jnp.int32)
    srcp = jnp.concatenate(
        [src, jnp.zeros((EP - E,), jnp.int32)]).reshape(NW, NBLK, BLK)
    dstp = jnp.concatenate(
        [dst, jnp.full((EP - E,), N, jnp.int32)]).reshape(NW, NBLK, BLK)
    x_pad = jnp.pad(x, ((0, NP - N), (0, 0)))
    bt = jnp.concatenate(
        [batch.astype(jnp.int32),
         jnp.full((NP - N,), G, jnp.int32)]).reshape(NRB, 1, RB)
    idr = jnp.arange(640, dtype=jnp.int32).reshape(5, 128)
    prm = jnp.pad(jnp.stack([b1, g1, be1, b2, g2, be2,
                             jnp.zeros_like(b1), jnp.zeros_like(b1)]),
                  ((0, 0), (0, 128 - H)))

    degp = _sc_deg(dst, idr)
    mm1 = _tc_mm(x_pad, W1)
    dinv = _tc_dinv(degp.reshape(2, 80, 128))
    dcol = dinv.reshape(NP)[:, None]

    h1p = _tc_scale(mm1, dcol)
    a1a, a1b = _sc_agg(h1p[:, :HH], h1p[:, HH:], srcp, dstp)
    out1, st1 = _tc_post(a1a.reshape(2, NP, HH), a1b.reshape(2, NP, HH),
                         h1p, dcol, prm, brow=0)
    h2p = _tc_bnmm(out1, st1, prm, W2, dcol, grow=1, berow=2)
    a2a, a2b = _sc_agg(h2p[:, :HH], h2p[:, HH:], srcp, dstp)
    out2, st2 = _tc_post(a2a.reshape(2, NP, HH), a2b.reshape(2, NP, HH),
                         h2p, dcol, prm, brow=3)
    return _tc_final(out2, st2, prm, bt)


# speedup vs baseline: 1.9629x; 1.0189x over previous
import functools

import jax
import jax.numpy as jnp
from jax import lax
from jax.experimental import pallas as pl
from jax.experimental.pallas import tpu as pltpu
from jax.experimental.pallas import tpu_sc as plsc

N = 10000
E = 320000
F = 128
H = 64
G = 16
EPS = 1e-5

NC, NS = 2, 16
NW = NC * NS
NP = 10240
EP = 327680
BLK = 128
NBLK = EP // (NW * BLK)
DPT = E // NW
RB = 2048
NRB = NP // RB

@functools.cache
def _sc_params():
    import dataclasses
    cp = pltpu.CompilerParams()
    if "needs_layout_passes" in pltpu.CompilerParams.__dataclass_fields__:
        cp = dataclasses.replace(cp, needs_layout_passes=False)
    if "use_tc_tiling_on_sc" in pltpu.CompilerParams.__dataclass_fields__:
        cp = dataclasses.replace(cp, use_tc_tiling_on_sc=False)
    return cp


@functools.cache
def _mesh():
    return plsc.VectorSubcoreMesh(
        core_axis_name="c", subcore_axis_name="s",
        num_cores=NC, num_subcores=NS)


def _sc_deg_body(dst_hbm, idr_hbm, out_hbm, idx_v, hist_v, idr_v, zv, acc_sh,
                 sem):
    c = lax.axis_index("c")
    s = lax.axis_index("s")
    wid = s * NC + c
    zero16 = jnp.zeros((16,), jnp.float32)

    @pl.loop(0, 640)
    def _zero_hist(i):
        hist_v[i, :] = zero16

    @pl.loop(0, 40)
    def _zero_zv(i):
        zv[i, :] = zero16

    pltpu.sync_copy(zv, acc_sh.at[pl.ds(s * 40, 40)])
    pltpu.sync_copy(dst_hbm.at[pl.ds(wid * DPT, DPT)], idx_v)
    pltpu.sync_copy(idr_hbm, idr_v)
    plsc.subcore_barrier()

    ones16 = jnp.ones((16,), jnp.float32)

    @pl.loop(0, DPT // 16)
    def _hist(i):
        nid = idx_v[pl.ds(i * 16, 16)]
        row = lax.shift_right_logical(nid, 4)
        col = lax.bitwise_and(nid, 15)
        plsc.addupdate_scatter(hist_v, [row, col], ones16)

    @pl.loop(0, 5)
    def _merge(j):
        pltpu.sync_copy(hist_v.at[pl.ds(j * 128, 128)],
                        acc_sh.at[idr_v.at[j]], add=True)

    plsc.subcore_barrier()
    pltpu.sync_copy(acc_sh.at[pl.ds(s * 40, 40)],
                    out_hbm.at[pl.ds(c * 640 + s * 40, 40)])


@jax.jit
def _sc_deg(dst, idr):
    return pl.kernel(
        _sc_deg_body,
        out_type=jax.ShapeDtypeStruct((2 * 640, 16), jnp.float32),
        mesh=_mesh(),
        compiler_params=_sc_params(),
        scratch_types=[
            pltpu.VMEM((DPT,), jnp.int32),
            pltpu.VMEM((640, 16), jnp.float32),
            pltpu.VMEM((5, 128), jnp.int32),
            pltpu.VMEM((40, 16), jnp.float32),
            pltpu.VMEM_SHARED((640, 16), jnp.float32),
            pltpu.SemaphoreType.DMA,
        ],
    )(dst, idr)


HH = H // 2


def _sc_agg_body(tabA, tabB, srcp_hbm, dstp_hbm, outA, outB,
                 sidx_v, didx_v, rows, semg, sems, acc_sh, tab_sh):
    c = lax.axis_index("c")
    s = lax.axis_index("s")
    wid = s * NC + c
    rpt = NP // NS
    NB = 4

    pltpu.sync_copy(srcp_hbm.at[wid], sidx_v)
    pltpu.sync_copy(dstp_hbm.at[wid], didx_v)

    for tab_hbm, out_hbm in ((tabA, outA), (tabB, outB)):
        pltpu.sync_copy(tab_hbm.at[pl.ds(s * rpt, rpt)],
                        tab_sh.at[pl.ds(s * rpt, rpt)])

        @pl.loop(0, BLK)
        def _zrow(i):
            @pl.loop(0, HH, step=16)
            def _zcol(k):
                rows[0, i, pl.ds(k, 16)] = jnp.zeros((16,), jnp.float32)

        @pl.loop(0, rpt, step=BLK)
        def _zacc(r):
            pltpu.sync_copy(rows.at[0], acc_sh.at[pl.ds(s * rpt + r, BLK)])

        plsc.subcore_barrier()

        def gather(k, b):
            pltpu.async_copy(tab_sh.at[sidx_v.at[k]], rows.at[b], semg[b])

        def wait_gather(k, b):
            pltpu.make_async_copy(tab_sh.at[sidx_v.at[k]], rows.at[b],
                                  semg[b]).wait()

        def scat(k, b):
            pltpu.async_copy(rows.at[b], acc_sh.at[didx_v.at[k]], sems[b],
                             add=True)

        def wait_scat(k, b):
            pltpu.make_async_copy(rows.at[b], acc_sh.at[didx_v.at[k]],
                                  sems[b]).wait()

        gather(0, 0)
        gather(1, 1)

        @pl.loop(0, NBLK, step=NB)
        def _edges(j):
            for b in range(NB):
                k = j + b
                gb = (b + 2) % NB

                @pl.when(k - 2 >= 0)
                def _():
                    wait_scat(k - 2, gb)

                @pl.when(k + 2 < NBLK)
                def _():
                    gather(k + 2, gb)

                wait_gather(k, b)
                scat(k, b)

        for k in (NBLK - 2, NBLK - 1):
            wait_scat(k, k % NB)

        plsc.subcore_barrier()
        pltpu.sync_copy(acc_sh.at[pl.ds(s * rpt, rpt)],
                        out_hbm.at[pl.ds(c * NP + s * rpt, rpt)])
        plsc.subcore_barrier()


@jax.jit
def _sc_agg(tabA, tabB, srcp, dstp):
    return pl.kernel(
        _sc_agg_body,
        out_type=(jax.ShapeDtypeStruct((2 * NP, HH), jnp.float32),
                  jax.ShapeDtypeStruct((2 * NP, HH), jnp.float32)),
        mesh=_mesh(),
        compiler_params=_sc_params(),
        scratch_types=[
            pltpu.VMEM((NBLK, BLK), jnp.int32),
            pltpu.VMEM((NBLK, BLK), jnp.int32),
            pltpu.VMEM((4, BLK, HH), jnp.float32),
            [pltpu.SemaphoreType.DMA] * 4,
            [pltpu.SemaphoreType.DMA] * 4,
            pltpu.VMEM_SHARED((NP, HH), jnp.float32),
            pltpu.VMEM_SHARED((NP, HH), jnp.float32),
        ],
    )(tabA, tabB, srcp, dstp)


def _mm_body(x_ref, w_ref, o_ref):
    o_ref[...] = jnp.dot(x_ref[...], w_ref[...],
                         preferred_element_type=jnp.float32)


def _tc_mm(x, w):
    m, k = x.shape
    _, n = w.shape
    return pl.pallas_call(
        _mm_body,
        grid=(m // RB,),
        in_specs=[pl.BlockSpec((RB, k), lambda i: (i, 0)),
                  pl.BlockSpec((k, n), lambda i: (0, 0))],
        out_specs=pl.BlockSpec((RB, n), lambda i: (i, 0)),
        out_shape=jax.ShapeDtypeStruct((m, n), jnp.float32),
    )(x, w)


def _dinv_body(dp_ref, o_ref):
    deg = dp_ref[0] + dp_ref[1] + 1.0
    r = lax.broadcasted_iota(jnp.int32, (80, 128), 0)
    cidx = lax.broadcasted_iota(jnp.int32, (80, 128), 1)
    nid = r * 128 + cidx
    o_ref[...] = jnp.where(nid < N, lax.rsqrt(deg), 0.0)


def _tc_dinv(degp):
    return pl.pallas_call(
        _dinv_body,
        out_shape=jax.ShapeDtypeStruct((80, 128), jnp.float32),
    )(degp)


def _scale_body(m_ref, d_ref, oa_ref, ob_ref):
    v = m_ref[...] * d_ref[...]
    oa_ref[...] = v[:, :HH]
    ob_ref[...] = v[:, HH:]


def _tc_scale(m, dcol):
    return pl.pallas_call(
        _scale_body,
        grid=(NRB,),
        in_specs=[pl.BlockSpec((RB, H), lambda i: (i, 0)),
                  pl.BlockSpec((RB, 1), lambda i: (i, 0))],
        out_specs=[pl.BlockSpec((RB, HH), lambda i: (i, 0)),
                   pl.BlockSpec((RB, HH), lambda i: (i, 0))],
        out_shape=[jax.ShapeDtypeStruct((NP, HH), jnp.float32),
                   jax.ShapeDtypeStruct((NP, HH), jnp.float32)],
    )(m, dcol)


def _post_body(aa0_ref, aa1_ref, ab0_ref, ab1_ref, hpa_ref, hpb_ref, d_ref,
               prm_ref, o_ref, st_ref, *, brow):
    i = pl.program_id(0)
    b = prm_ref[brow:brow + 1, :H]
    agg = jnp.concatenate(
        [aa0_ref[...] + aa1_ref[...] + hpa_ref[...],
         ab0_ref[...] + ab1_ref[...] + hpb_ref[...]], axis=1)
    v = agg * d_ref[...] + b
    rid = lax.broadcasted_iota(jnp.int32, (RB, 1), 0) + i * RB
    v = jnp.where(rid < N, v, 0.0)
    o_ref[...] = v
    srow = jnp.sum(v, axis=0, keepdims=True)
    qrow = jnp.sum(v * v, axis=0, keepdims=True)
    st = jnp.concatenate(
        [srow, qrow, jnp.zeros((6, H), jnp.float32)], axis=0)

    @pl.when(i == 0)
    def _():
        st_ref[...] = st

    @pl.when(i > 0)
    def _():
        st_ref[...] = st_ref[...] + st


def _tc_post(accpa, accpb, hpa, hpb, dcol, prm, brow):
    return pl.pallas_call(
        functools.partial(_post_body, brow=brow),
        grid=(NRB,),
        in_specs=[pl.BlockSpec((RB, HH), lambda i: (i, 0)),
                  pl.BlockSpec((RB, HH), lambda i: (NRB + i, 0)),
                  pl.BlockSpec((RB, HH), lambda i: (i, 0)),
                  pl.BlockSpec((RB, HH), lambda i: (NRB + i, 0)),
                  pl.BlockSpec((RB, HH), lambda i: (i, 0)),
                  pl.BlockSpec((RB, HH), lambda i: (i, 0)),
                  pl.BlockSpec((RB, 1), lambda i: (i, 0)),
                  pl.BlockSpec((8, 128), lambda i: (0, 0))],
        out_specs=[pl.BlockSpec((RB, H), lambda i: (i, 0)),
                   pl.BlockSpec((8, H), lambda i: (0, 0))],
        out_shape=[jax.ShapeDtypeStruct((NP, H), jnp.float32),
                   jax.ShapeDtypeStruct((8, H), jnp.float32)],
    )(accpa, accpa, accpb, accpb, hpa, hpb, dcol, prm)


def _bnmm_body(o1_ref, st_ref, prm_ref, w_ref, d_ref, oa_ref, ob_ref, *,
               grow, berow):
    mu = st_ref[0:1, :] * (1.0 / N)
    var = st_ref[1:2, :] * (1.0 / N) - mu * mu
    istd = lax.rsqrt(var + EPS)
    g = prm_ref[grow:grow + 1, :H]
    be = prm_ref[berow:berow + 1, :H]
    h = jnp.maximum((o1_ref[...] - mu) * istd * g + be, 0.0)
    v = jnp.dot(h, w_ref[...],
                preferred_element_type=jnp.float32) * d_ref[...]
    oa_ref[...] = v[:, :HH]
    ob_ref[...] = v[:, HH:]


def _tc_bnmm(o1, st, prm, w, dcol, grow, berow):
    return pl.pallas_call(
        functools.partial(_bnmm_body, grow=grow, berow=berow),
        grid=(NRB,),
        in_specs=[pl.BlockSpec((RB, H), lambda i: (i, 0)),
                  pl.BlockSpec((8, H), lambda i: (0, 0)),
                  pl.BlockSpec((8, 128), lambda i: (0, 0)),
                  pl.BlockSpec((H, H), lambda i: (0, 0)),
                  pl.BlockSpec((RB, 1), lambda i: (i, 0))],
        out_specs=[pl.BlockSpec((RB, HH), lambda i: (i, 0)),
                   pl.BlockSpec((RB, HH), lambda i: (i, 0))],
        out_shape=[jax.ShapeDtypeStruct((NP, HH), jnp.float32),
                   jax.ShapeDtypeStruct((NP, HH), jnp.float32)],
    )(o1, st, prm, w, dcol)


def _final_body(o2_ref, st_ref, prm_ref, bt_ref, o_ref, cnt_ref):
    i = pl.program_id(0)
    mu = st_ref[0:1, :] * (1.0 / N)
    var = st_ref[1:2, :] * (1.0 / N) - mu * mu
    istd = lax.rsqrt(var + EPS)
    g = prm_ref[4:5, :H]
    be = prm_ref[5:6, :H]
    h = jnp.maximum((o2_ref[...] - mu) * istd * g + be, 0.0)
    b = bt_ref[0, 0, :]
    gid = lax.broadcasted_iota(jnp.int32, (G, RB), 0)
    oh = jnp.where(gid == b[None, :], 1.0, 0.0)
    ps = jnp.dot(oh, h, preferred_element_type=jnp.float32)
    cnt = jnp.broadcast_to(jnp.sum(oh, axis=1, keepdims=True), (G, H))

    @pl.when(i == 0)
    def _():
        o_ref[...] = ps
        cnt_ref[...] = cnt

    @pl.when(i > 0)
    def _():
        o_ref[...] = o_ref[...] + ps
        cnt_ref[...] = cnt_ref[...] + cnt

    @pl.when(i == NRB - 1)
    def _():
        o_ref[...] = o_ref[...] / jnp.maximum(cnt_ref[...], 1.0)


def _tc_final(o2, st, prm, bt):
    return pl.pallas_call(
        _final_body,
        grid=(NRB,),
        in_specs=[pl.BlockSpec((RB, H), lambda i: (i, 0)),
                  pl.BlockSpec((8, H), lambda i: (0, 0)),
                  pl.BlockSpec((8, 128), lambda i: (0, 0)),
                  pl.BlockSpec((1, 1, RB), lambda i: (i, 0, 0))],
        out_specs=pl.BlockSpec((G, H), lambda i: (0, 0)),
        out_shape=jax.ShapeDtypeStruct((G, H), jnp.float32),
        scratch_shapes=[pltpu.VMEM((G, H), jnp.float32)],
    )(o2, st, prm, bt)


def kernel(x, ei, batch, W1, b1, g1, be1, W2, b2, g2, be2):
    src = ei[0].astype(jnp.int32)
    dst = ei[1].astype(jnp.int32)
    srcp = jnp.concatenate(
        [src, jnp.zeros((EP - E,), jnp.int32)]).reshape(NW, NBLK, BLK)
    dstp = jnp.concatenate(
        [dst, jnp.full((EP - E,), N, jnp.int32)]).reshape(NW, NBLK, BLK)
    x_pad = jnp.pad(x, ((0, NP - N), (0, 0)))
    bt = jnp.concatenate(
        [batch.astype(jnp.int32),
         jnp.full((NP - N,), G, jnp.int32)]).reshape(NRB, 1, RB)
    idr = jnp.arange(640, dtype=jnp.int32).reshape(5, 128)
    prm = jnp.pad(jnp.stack([b1, g1, be1, b2, g2, be2,
                             jnp.zeros_like(b1), jnp.zeros_like(b1)]),
                  ((0, 0), (0, 128 - H)))

    degp = _sc_deg(dst, idr)
    mm1 = _tc_mm(x_pad, W1)
    dinv = _tc_dinv(degp.reshape(2, 80, 128))
    dcol = dinv.reshape(NP)[:, None]

    h1pa, h1pb = _tc_scale(mm1, dcol)
    a1a, a1b = _sc_agg(h1pa, h1pb, srcp, dstp)
    out1, st1 = _tc_post(a1a, a1b, h1pa, h1pb, dcol, prm, brow=0)
    h2pa, h2pb = _tc_bnmm(out1, st1, prm, W2, dcol, grow=1, berow=2)
    a2a, a2b = _sc_agg(h2pa, h2pb, srcp, dstp)
    out2, st2 = _tc_post(a2a, a2b, h2pa, h2pb, dcol, prm, brow=3)
    return _tc_final(out2, st2, prm, bt)
